# baseline (speedup 1.0000x reference)
import functools
import math

import jax
import jax.numpy as jnp
from jax import lax
from jax.experimental import pallas as pl
from jax.experimental.pallas import tpu as pltpu
from jax.experimental.pallas import tpu_sc as plsc

D_MODEL = 2048
HIDDEN = 256
NUM_EXPERTS = 64
TOP_K = 2
N_TOK = 16384

TILE = 2048

_INV_SQRT2 = 1.0 / math.sqrt(2.0)

NC = 2
NS = 16
NW = NC * NS
TOK_PER_W = N_TOK // NW
L = 16
GROUPS = TOK_PER_W // L


def _logits_kernel(x_ref, w1_ref, b1_ref, w2_ref, b2_ref, lt_out_ref):
    h = jnp.dot(x_ref[...], w1_ref[...], preferred_element_type=jnp.float32)
    h = h + b1_ref[...]
    h = 0.5 * h * (1.0 + jax.lax.erf(h * _INV_SQRT2))
    logits = jnp.dot(h, w2_ref[...], preferred_element_type=jnp.float32)
    logits = logits + b2_ref[...]
    lt_out_ref[...] = logits.T


def _tc_logits(x, W1, b1r, W2, b2r):
    grid = (N_TOK // TILE,)
    return pl.pallas_call(
        _logits_kernel,
        grid=grid,
        in_specs=[
            pl.BlockSpec((TILE, D_MODEL), lambda i: (i, 0)),
            pl.BlockSpec((D_MODEL, HIDDEN), lambda i: (0, 0)),
            pl.BlockSpec((1, HIDDEN), lambda i: (0, 0)),
            pl.BlockSpec((HIDDEN, NUM_EXPERTS), lambda i: (0, 0)),
            pl.BlockSpec((1, NUM_EXPERTS), lambda i: (0, 0)),
        ],
        out_specs=pl.BlockSpec((NUM_EXPERTS, TILE), lambda i: (0, i)),
        out_shape=jax.ShapeDtypeStruct((NUM_EXPERTS, N_TOK), jnp.float32),
        compiler_params=pltpu.CompilerParams(
            dimension_semantics=("arbitrary",),
        ),
    )(x, W1, b1r, W2, b2r)


@functools.partial(
    pl.kernel,
    mesh=plsc.VectorSubcoreMesh(core_axis_name="c", subcore_axis_name="s"),
    out_type=[
        jax.ShapeDtypeStruct((N_TOK,), jnp.float32),
        jax.ShapeDtypeStruct((N_TOK,), jnp.float32),
        jax.ShapeDtypeStruct((N_TOK,), jnp.int32),
        jax.ShapeDtypeStruct((N_TOK,), jnp.int32),
    ],
    scratch_types=[
        pltpu.VMEM((NUM_EXPERTS, TOK_PER_W), jnp.float32),
        pltpu.VMEM((TOK_PER_W,), jnp.float32),
        pltpu.VMEM((TOK_PER_W,), jnp.float32),
        pltpu.VMEM((TOK_PER_W,), jnp.int32),
        pltpu.VMEM((TOK_PER_W,), jnp.int32),
        pltpu.SemaphoreType.DMA,
    ],
)
def _sc_gating(lt_hbm, w1_out, w2_out, i1_out, i2_out,
               lg_v, w1_v, w2_v, i1_v, i2_v, sem):
    wid = lax.axis_index("s") * NC + lax.axis_index("c")
    base = wid * TOK_PER_W
    pltpu.async_copy(lt_hbm.at[:, pl.ds(base, TOK_PER_W)], lg_v, sem).wait()

    def group_body(g, carry):
        neg = jnp.full((L,), -jnp.inf, jnp.float32)
        zero = jnp.zeros((L,), jnp.float32)

        def expert_body(e, st):
            m1, m2, r1, r2 = st
            v = lg_v[e, pl.ds(g * L, L)]
            ef = e.astype(jnp.float32)
            gt1 = v > m1
            gt2 = v > m2
            n_m2 = jnp.where(gt1, m1, jnp.where(gt2, v, m2))
            n_r2 = jnp.where(gt1, r1, jnp.where(gt2, ef, r2))
            n_m1 = jnp.where(gt1, v, m1)
            n_r1 = jnp.where(gt1, ef, r1)
            return (n_m1, n_m2, n_r1, n_r2)

        m1, m2, r1, r2 = lax.fori_loop(
            0, NUM_EXPERTS, expert_body, (neg, neg, zero, zero))
        e2 = jnp.exp(m2 - m1)
        d = 1.0 + e2
        w1_v[pl.ds(g * L, L)] = 1.0 / d
        w2_v[pl.ds(g * L, L)] = e2 / d
        i1_v[pl.ds(g * L, L)] = r1.astype(jnp.int32)
        i2_v[pl.ds(g * L, L)] = r2.astype(jnp.int32)
        return carry

    lax.fori_loop(0, GROUPS, group_body, 0)

    pltpu.sync_copy(w1_v, w1_out.at[pl.ds(base, TOK_PER_W)])
    pltpu.sync_copy(w2_v, w2_out.at[pl.ds(base, TOK_PER_W)])
    pltpu.sync_copy(i1_v, i1_out.at[pl.ds(base, TOK_PER_W)])
    pltpu.sync_copy(i2_v, i2_out.at[pl.ds(base, TOK_PER_W)])


@jax.jit
def kernel(x, W1, b1, W2, b2):
    b1r = b1.reshape(1, HIDDEN)
    b2r = b2.reshape(1, NUM_EXPERTS)
    lt = _tc_logits(x, W1, b1r, W2, b2r)
    w1c, w2c, i1c, i2c = _sc_gating(lt)
    weights = jnp.stack([w1c, w2c], axis=1)
    idx = jnp.stack([i1c, i2c], axis=1)
    return (weights, idx)


# speedup vs baseline: 1.2366x; 1.2366x over previous
import functools
import math

import jax
import jax.numpy as jnp
from jax.experimental import pallas as pl
from jax.experimental.pallas import tpu as pltpu

D_MODEL = 2048
HIDDEN = 256
NUM_EXPERTS = 64
TOP_K = 2
N_TOK = 16384

TILE = 2048

_INV_SQRT2 = 1.0 / math.sqrt(2.0)


def _fused_gating_kernel(x_ref, w1_ref, b1_ref, w2_ref, b2_ref,
                         w_out_ref, i_out_ref):
    h = jnp.dot(x_ref[...], w1_ref[...], preferred_element_type=jnp.float32)
    h = h + b1_ref[...]
    h = 0.5 * h * (1.0 + jax.lax.erf(h * _INV_SQRT2))
    logits = jnp.dot(h, w2_ref[...], preferred_element_type=jnp.float32)
    logits = logits + b2_ref[...]

    col = jax.lax.broadcasted_iota(jnp.int32, logits.shape, 1)
    revcol = (NUM_EXPERTS - 1 - col).astype(jnp.float32)
    m1 = jnp.max(logits, axis=1, keepdims=True)
    r1 = jnp.max(jnp.where(logits == m1, revcol, -1.0), axis=1,
                 keepdims=True)
    i1 = (NUM_EXPERTS - 1) - r1.astype(jnp.int32)
    masked = jnp.where(col == i1, -jnp.inf, logits)
    m2 = jnp.max(masked, axis=1, keepdims=True)
    r2 = jnp.max(jnp.where(masked == m2, revcol, -1.0), axis=1,
                 keepdims=True)
    i2 = (NUM_EXPERTS - 1) - r2.astype(jnp.int32)

    e2 = jnp.exp(m2 - m1)
    denom = 1.0 + e2
    w_out_ref[...] = jnp.concatenate([1.0 / denom, e2 / denom], axis=1)
    i_out_ref[...] = jnp.concatenate([i1, i2], axis=1)


@jax.jit
def kernel(x, W1, b1, W2, b2):
    b1r = b1.reshape(1, HIDDEN)
    b2r = b2.reshape(1, NUM_EXPERTS)
    grid = (N_TOK // TILE,)
    weights, topk_i = pl.pallas_call(
        _fused_gating_kernel,
        grid=grid,
        in_specs=[
            pl.BlockSpec((TILE, D_MODEL), lambda i: (i, 0)),
            pl.BlockSpec((D_MODEL, HIDDEN), lambda i: (0, 0)),
            pl.BlockSpec((1, HIDDEN), lambda i: (0, 0)),
            pl.BlockSpec((HIDDEN, NUM_EXPERTS), lambda i: (0, 0)),
            pl.BlockSpec((1, NUM_EXPERTS), lambda i: (0, 0)),
        ],
        out_specs=[
            pl.BlockSpec((TILE, TOP_K), lambda i: (i, 0)),
            pl.BlockSpec((TILE, TOP_K), lambda i: (i, 0)),
        ],
        out_shape=[
            jax.ShapeDtypeStruct((N_TOK, TOP_K), jnp.float32),
            jax.ShapeDtypeStruct((N_TOK, TOP_K), jnp.int32),
        ],
        compiler_params=pltpu.CompilerParams(
            dimension_semantics=("arbitrary",),
        ),
    )(x, W1, b1r, W2, b2r)
    return (weights, topk_i)
